# Initial kernel scaffold; baseline (speedup 1.0000x reference)
import jax
import jax.numpy as jnp
from jax.experimental import pallas as pl


def kernel(x, emb):
    raise NotImplementedError("write your pallas kernel here")


# speedup vs baseline: 7.1747x; 7.1747x over previous
import functools

import jax
import jax.numpy as jnp
from jax import lax
from jax.experimental import pallas as pl
from jax.experimental.pallas import tpu as pltpu
from jax.experimental.pallas import tpu_sc as plsc

QL = 1000
QE = 32
CH = 26
B = 4096
L = 20

ROWS = B * CH * L
NW = 32
RPW = ROWS // NW
CHUNK = 2 * CH * L
NCHUNK = RPW // CHUNK
NVEC = CHUNK // 16


def _sc_body(x_hbm, emb_hbm, off_hbm, out_hbm, off_v, idx_v, gidx_v, rows_v, sem):
    wid = lax.axis_index("s") * 2 + lax.axis_index("c")
    base = wid * RPW
    pltpu.sync_copy(off_hbm, off_v)

    def chunk_body(ci, carry):
        row0 = base + ci * CHUNK
        pltpu.sync_copy(x_hbm.at[pl.ds(row0, CHUNK)], idx_v)
        for i in range(NVEC):
            sl = pl.ds(i * 16, 16)
            gidx_v[sl] = idx_v[sl] + off_v[sl]
        pltpu.async_copy(emb_hbm.at[gidx_v], rows_v, sem).wait()
        pltpu.sync_copy(rows_v, out_hbm.at[pl.ds(row0, CHUNK)])
        return carry

    lax.fori_loop(0, NCHUNK, chunk_body, 0)


@functools.partial(
    pl.kernel,
    out_type=jax.ShapeDtypeStruct((ROWS, QE), jnp.float32),
    mesh=plsc.VectorSubcoreMesh(core_axis_name="c", subcore_axis_name="s"),
    scratch_types=[
        pltpu.VMEM((CHUNK,), jnp.int32),
        pltpu.VMEM((CHUNK,), jnp.int32),
        pltpu.VMEM((CHUNK,), jnp.int32),
        pltpu.VMEM((CHUNK, QE), jnp.float32),
        pltpu.SemaphoreType.DMA,
    ],
    compiler_params=pltpu.CompilerParams(use_tc_tiling_on_sc=False),
)
def _sc_gather(x_hbm, emb_hbm, off_hbm, out_hbm, off_v, idx_v, gidx_v, rows_v, sem):
    _sc_body(x_hbm, emb_hbm, off_hbm, out_hbm, off_v, idx_v, gidx_v, rows_v, sem)


def kernel(x, emb):
    x_flat = x.reshape(ROWS).astype(jnp.int32)
    emb_flat = emb.reshape(CH * QL, QE)
    off = jnp.tile(jnp.repeat(jnp.arange(CH, dtype=jnp.int32) * QL, L), 2)
    out = _sc_gather(x_flat, emb_flat, off)
    return out.reshape(B, CH, L, QE)


# speedup vs baseline: 7.4404x; 1.0370x over previous
import functools

import jax
import jax.numpy as jnp
from jax import lax
from jax.experimental import pallas as pl
from jax.experimental.pallas import tpu as pltpu
from jax.experimental.pallas import tpu_sc as plsc

QL = 1000
QE = 32
CH = 26
B = 4096
L = 20

ROWS = B * CH * L
NW = 32
RPW = ROWS // NW
CHUNK = 2 * CH * L
NCHUNK = RPW // CHUNK
NVEC = CHUNK // 16


def _sc_body(x_hbm, emb_hbm, off_hbm, out_hbm, off_v, idx_v, gidx_v,
             rows0_v, rows1_v, gsem, wsem0, wsem1):
    wid = lax.axis_index("s") * 2 + lax.axis_index("c")
    base = wid * RPW
    rows = (rows0_v, rows1_v)
    wsems = (wsem0, wsem1)
    pltpu.sync_copy(off_hbm, off_v)

    def do_chunk(ci, p, first):
        row0 = base + ci * CHUNK
        pltpu.sync_copy(x_hbm.at[pl.ds(row0, CHUNK)], idx_v)
        for i in range(NVEC):
            sl = pl.ds(i * 16, 16)
            gidx_v[sl] = idx_v[sl] + off_v[sl]
        if not first:
            pltpu.make_async_copy(rows[p], out_hbm.at[pl.ds(row0, CHUNK)],
                                  wsems[p]).wait()
        pltpu.async_copy(emb_hbm.at[gidx_v], rows[p], gsem).wait()
        cp = pltpu.make_async_copy(rows[p], out_hbm.at[pl.ds(row0, CHUNK)],
                                   wsems[p])
        cp.start()

    do_chunk(0, 0, True)
    do_chunk(1, 1, True)

    def pair_body(j, carry):
        do_chunk(2 * j, 0, False)
        do_chunk(2 * j + 1, 1, False)
        return carry

    lax.fori_loop(1, NCHUNK // 2, pair_body, 0)

    for p in range(2):
        pltpu.make_async_copy(rows[p], out_hbm.at[pl.ds(base, CHUNK)],
                              wsems[p]).wait()


@functools.partial(
    pl.kernel,
    out_type=jax.ShapeDtypeStruct((ROWS, QE), jnp.float32),
    mesh=plsc.VectorSubcoreMesh(core_axis_name="c", subcore_axis_name="s"),
    scratch_types=[
        pltpu.VMEM((CHUNK,), jnp.int32),
        pltpu.VMEM((CHUNK,), jnp.int32),
        pltpu.VMEM((CHUNK,), jnp.int32),
        pltpu.VMEM((CHUNK, QE), jnp.float32),
        pltpu.VMEM((CHUNK, QE), jnp.float32),
        pltpu.SemaphoreType.DMA,
        pltpu.SemaphoreType.DMA,
        pltpu.SemaphoreType.DMA,
    ],
    compiler_params=pltpu.CompilerParams(use_tc_tiling_on_sc=False),
)
def _sc_gather(x_hbm, emb_hbm, off_hbm, out_hbm, off_v, idx_v, gidx_v,
               rows0_v, rows1_v, gsem, wsem0, wsem1):
    _sc_body(x_hbm, emb_hbm, off_hbm, out_hbm, off_v, idx_v, gidx_v,
             rows0_v, rows1_v, gsem, wsem0, wsem1)


def kernel(x, emb):
    x_flat = x.reshape(ROWS).astype(jnp.int32)
    emb_flat = emb.reshape(CH * QL, QE)
    off = jnp.tile(jnp.repeat(jnp.arange(CH, dtype=jnp.int32) * QL, L), 2)
    out = _sc_gather(x_flat, emb_flat, off)
    return out.reshape(B, CH, L, QE)
